# baseline (speedup 1.0000x reference)
import jax
import jax.numpy as jnp
from jax import lax
from jax.experimental import pallas as pl
from jax.experimental.pallas import tpu as pltpu
from jax.experimental.pallas import tpu_sc as plsc

N, E, F, H, C, G = 10000, 320000, 128, 128, 2, 128
NC, NS, LANES = 2, 16, 16
NP = 10240
RPT = NP // NS
FH = H // NC
CH = 128
CPT = 160
EPAD = NS * CPT * CH
GP = G + 8
MW = (G * FH) // NS
RB = 2048
PCH = 40
NBUF = 3
GL = 2
IL = 3
IBUF = 8

_mesh = plsc.VectorSubcoreMesh(
    core_axis_name="c", subcore_axis_name="s", num_cores=NC, num_subcores=NS)


def _k1_body(dst_ref, batch_ref, x_ref, degp_out, xr_out,
             dstbuf, degbuf, batchbuf, histbuf, rootsbuf, xrbuf):
    c = lax.axis_index("c")
    s = lax.axis_index("s")
    ept = E // NS

    @pl.when(c == 0)
    def _():
        def zero_deg(i, carry):
            degbuf[pl.ds(i * LANES, LANES)] = jnp.zeros((LANES,), jnp.float32)
            return carry
        lax.fori_loop(0, NP // LANES, zero_deg, 0)

        pltpu.sync_copy(dst_ref.at[pl.ds(s * ept, ept)], dstbuf)
        ones = jnp.ones((LANES,), jnp.float32)

        def acc_deg(i, carry):
            idx = dstbuf[pl.ds(i * LANES, LANES)]
            plsc.addupdate_scatter(degbuf, [idx], ones)
            return carry
        lax.fori_loop(0, ept // LANES, acc_deg, 0)
        pltpu.sync_copy(degbuf, degp_out.at[pl.ds(s * NP, NP)])

    @pl.when((c == 1) & (s == 0))
    def _():
        for k in range(G // LANES):
            histbuf[pl.ds(k * LANES, LANES)] = jnp.zeros((LANES,), jnp.int32)
        pltpu.sync_copy(batch_ref, batchbuf)
        ones_i = jnp.ones((LANES,), jnp.int32)

        def acc_hist(i, carry):
            idx = batchbuf[pl.ds(i * LANES, LANES)]
            plsc.addupdate_scatter(histbuf, [idx], ones_i)
            return carry
        lax.fori_loop(0, N // LANES, acc_hist, 0)

        carry = jnp.int32(0)
        for k in range(G // LANES):
            v = histbuf[pl.ds(k * LANES, LANES)]
            inc = plsc.cumsum(v)
            excl = inc - v + carry
            rootsbuf[pl.ds(k * LANES, LANES)] = jnp.minimum(excl, N - 1)
            carry = carry + jnp.sum(v)
        pltpu.sync_copy(x_ref.at[rootsbuf], xrbuf)
        pltpu.sync_copy(xrbuf, xr_out)


_k1 = pl.kernel(
    _k1_body,
    out_type=[jax.ShapeDtypeStruct((NS * NP,), jnp.float32),
              jax.ShapeDtypeStruct((G, F), jnp.float32)],
    mesh=_mesh,
    compiler_params=pltpu.CompilerParams(needs_layout_passes=False, use_tc_tiling_on_sc=False),
    scratch_types=[
        pltpu.VMEM((E // NS,), jnp.int32),
        pltpu.VMEM((NP,), jnp.float32),
        pltpu.VMEM((N,), jnp.int32),
        pltpu.VMEM((G,), jnp.int32),
        pltpu.VMEM((G,), jnp.int32),
        pltpu.VMEM((G, F), jnp.float32),
    ],
)


def _prop_core(c, s, glo_ref, ghi_ref, src_ref, dst_ref,
               srcring, dstring, rowsbufs, sg, ss, si, sd, shared_acc,
               shared_g):
    rows = pl.ds(s * RPT, RPT)

    def run_core(gref):
        pltpu.sync_copy(gref.at[rows], shared_acc.at[rows])
        pltpu.sync_copy(gref.at[rows], shared_g.at[rows])
        plsc.subcore_barrier()

        def i_start(j):
            b = lax.rem(j, IBUF)
            pltpu.async_copy(src_ref.at[s * CPT + j], srcring.at[b], si.at[b])
            pltpu.async_copy(dst_ref.at[s * CPT + j], dstring.at[b], sd.at[b])

        def i_wait(j):
            b = lax.rem(j, IBUF)
            pltpu.make_async_copy(
                src_ref.at[s * CPT + j], srcring.at[b], si.at[b]).wait()
            pltpu.make_async_copy(
                dst_ref.at[s * CPT + j], dstring.at[b], sd.at[b]).wait()

        def g_start(j):
            b = lax.rem(j, NBUF)
            bi = lax.rem(j, IBUF)
            pltpu.async_copy(shared_g.at[srcring.at[bi]], rowsbufs.at[b],
                             sg.at[b])

        def g_wait(j):
            b = lax.rem(j, NBUF)
            bi = lax.rem(j, IBUF)
            pltpu.make_async_copy(
                shared_g.at[srcring.at[bi]], rowsbufs.at[b], sg.at[b]).wait()

        def s_start(j):
            b = lax.rem(j, NBUF)
            bi = lax.rem(j, IBUF)
            pltpu.async_copy(rowsbufs.at[b], shared_acc.at[dstring.at[bi]],
                             ss.at[b], add=True)

        def s_wait(j):
            b = lax.rem(j, NBUF)
            bi = lax.rem(j, IBUF)
            pltpu.make_async_copy(
                rowsbufs.at[b], shared_acc.at[dstring.at[bi]], ss.at[b]).wait()

        for j in range(IL):
            i_start(jnp.int32(j))
        for j in range(GL):
            i_wait(jnp.int32(j))
            g_start(jnp.int32(j))

        def step(i, carry):
            @pl.when(i + IL < CPT)
            def _():
                i_start(i + IL)

            @pl.when(i + GL < CPT)
            def _():
                @pl.when(i + GL >= NBUF)
                def _():
                    s_wait(i + GL - NBUF)
                i_wait(i + GL)
                g_start(i + GL)
            g_wait(i)
            s_start(i)
            return carry
        lax.fori_loop(0, CPT, step, 0)
        for k in range(NBUF):
            s_wait(jnp.int32(CPT - NBUF + k))

    @pl.when(c == 0)
    def _():
        run_core(glo_ref)

    @pl.when(c == 1)
    def _():
        run_core(ghi_ref)
    plsc.subcore_barrier()


def _k3_body(glo_ref, ghi_ref, src_ref, dst_ref, acclo_out, acchi_out,
             srcring, dstring, rowsbufs, sg, ss, si, sd, shared_acc,
             shared_g):
    c = lax.axis_index("c")
    s = lax.axis_index("s")
    _prop_core(c, s, glo_ref, ghi_ref, src_ref, dst_ref,
               srcring, dstring, rowsbufs, sg, ss, si, sd, shared_acc,
               shared_g)
    rows = pl.ds(s * RPT, RPT)

    @pl.when(c == 0)
    def _():
        pltpu.sync_copy(shared_acc.at[rows], acclo_out.at[rows])

    @pl.when(c == 1)
    def _():
        pltpu.sync_copy(shared_acc.at[rows], acchi_out.at[rows])


_prop_scratch = [
    pltpu.VMEM((IBUF, CH), jnp.int32),
    pltpu.VMEM((IBUF, CH), jnp.int32),
    pltpu.VMEM((NBUF, CH, FH), jnp.float32),
    pltpu.SemaphoreType.DMA((NBUF,)),
    pltpu.SemaphoreType.DMA((NBUF,)),
    pltpu.SemaphoreType.DMA((IBUF,)),
    pltpu.SemaphoreType.DMA((IBUF,)),
    pltpu.VMEM_SHARED((NP, FH), jnp.float32),
    pltpu.VMEM_SHARED((NP, FH), jnp.float32),
]

_k3 = pl.kernel(
    _k3_body,
    out_type=[jax.ShapeDtypeStruct((NP, FH), jnp.float32),
              jax.ShapeDtypeStruct((NP, FH), jnp.float32)],
    mesh=_mesh,
    compiler_params=pltpu.CompilerParams(needs_layout_passes=False, use_tc_tiling_on_sc=False),
    scratch_types=list(_prop_scratch),
)


def _k5_body(glo_ref, ghi_ref, src_ref, dst_ref, batch_ref, dis_ref, b2_ref,
             pool_out,
             srcring, dstring, rowsbufs, sg, ss, si, sd, shared_acc,
             shared_g,
             chunkbuf, batchv, disv, b2v, poolbuf, mtmp):
    c = lax.axis_index("c")
    s = lax.axis_index("s")
    _prop_core(c, s, glo_ref, ghi_ref, src_ref, dst_ref,
               srcring, dstring, rowsbufs, sg, ss, si, sd, shared_acc,
               shared_g)

    rows = pl.ds(s * RPT, RPT)
    pltpu.sync_copy(batch_ref.at[rows], batchv)
    pltpu.sync_copy(dis_ref.at[rows], disv)
    pltpu.sync_copy(b2_ref.at[c], b2v)
    ninf = jnp.full((LANES,), -jnp.inf, jnp.float32)

    def zero_pool(r, carry):
        for j in range(FH // LANES):
            poolbuf[r, pl.ds(j * LANES, LANES)] = ninf
        return carry
    lax.fori_loop(0, GP, zero_pool, 0)

    iota = lax.iota(jnp.int32, LANES)
    zi = jnp.zeros(( LANES,), jnp.int32)
    cols = [(j * LANES) + iota for j in range(FH // LANES)]
    b2vs = [b2v[0, pl.ds(j * LANES, LANES)] for j in range(FH // LANES)]

    def flush(cb, rms):
        for j in range(FH // LANES):
            old = plsc.load_gather(poolbuf, [cb, cols[j]])
            plsc.store_scatter(poolbuf, [cb, cols[j]],
                               jnp.maximum(old, rms[j]))

    ninf4 = [jnp.full((LANES,), -jnp.inf, jnp.float32)] * (FH // LANES)

    def pool_chunk(cc, carry):
        pltpu.sync_copy(shared_acc.at[pl.ds(s * RPT + cc * PCH, PCH)], chunkbuf)

        def row_step(rr, rcarry):
            cb = rcarry[0]
            rms = list(rcarry[1:])
            r = cc * PCH + rr
            rsplat = zi + r
            bvec = plsc.load_gather(batchv, [rsplat])
            dvec = plsc.load_gather(disv, [rsplat])
            changed = jnp.max(jnp.where(bvec != cb, 1, 0)) > 0
            valid = jnp.max(cb) >= 0

            @pl.when(changed & valid)
            def _():
                flush(cb, rms)
            hs = []
            for j in range(FH // LANES):
                v = chunkbuf[rr, pl.ds(j * LANES, LANES)]
                hs.append(jnp.maximum(v * dvec + b2vs[j], 0.0))
            rms = [jnp.where(changed, hs[j], jnp.maximum(rms[j], hs[j]))
                   for j in range(FH // LANES)]
            return (bvec, *rms)
        return lax.fori_loop(0, PCH, row_step, carry)
    fin = lax.fori_loop(0, RPT // PCH, pool_chunk,
                        (zi - 1, *ninf4))
    flush(fin[0], list(fin[1:]))

    pltpu.sync_copy(poolbuf.at[pl.ds(0, G)], shared_g.at[pl.ds(s * G, G)])
    plsc.subcore_barrier()
    gsl = pl.ds(s * (G // NS), G // NS)
    pltpu.sync_copy(shared_g.at[pl.ds(0 * G + s * (G // NS), G // NS)], mtmp)
    acc_vecs = [mtmp[r, pl.ds(j * LANES, LANES)]
                for r in range(G // NS) for j in range(FH // LANES)]
    for k in range(1, NS):
        pltpu.sync_copy(shared_g.at[pl.ds(k * G + s * (G // NS), G // NS)],
                        mtmp)
        vi = 0
        for r in range(G // NS):
            for j in range(FH // LANES):
                acc_vecs[vi] = jnp.maximum(
                    acc_vecs[vi], mtmp[r, pl.ds(j * LANES, LANES)])
                vi += 1
    vi = 0
    for r in range(G // NS):
        for j in range(FH // LANES):
            mtmp[r, pl.ds(j * LANES, LANES)] = acc_vecs[vi]
            vi += 1
    pltpu.sync_copy(mtmp, pool_out.at[pl.ds(c * G + s * (G // NS), G // NS)])


_k5 = pl.kernel(
    _k5_body,
    out_type=[jax.ShapeDtypeStruct((NC * G, FH), jnp.float32)],
    mesh=_mesh,
    compiler_params=pltpu.CompilerParams(needs_layout_passes=False, use_tc_tiling_on_sc=False),
    scratch_types=list(_prop_scratch) + [
        pltpu.VMEM((PCH, FH), jnp.float32),
        pltpu.VMEM((RPT,), jnp.int32),
        pltpu.VMEM((RPT,), jnp.float32),
        pltpu.VMEM((1, FH), jnp.float32),
        pltpu.VMEM((GP, FH), jnp.float32),
        pltpu.VMEM((G // NS, FH), jnp.float32),
    ],
)


def _k2_body(x_ref, w1_ref, degp_ref, xr_ref, wroot_ref, broot_ref,
             glo_ref, ghi_ref, dis_ref, hr_ref):
    m = lax.dot_general(x_ref[...], w1_ref[...], (((1,), (1,)), ((), ())),
                        preferred_element_type=jnp.float32)
    deg = lax.dot_general(degp_ref[...], jnp.ones((NS, 1), jnp.float32),
                          (((0,), (0,)), ((), ())),
                          preferred_element_type=jnp.float32)
    dis = lax.rsqrt(deg + 1.0)
    g1 = m * dis
    glo_ref[...] = g1[:, :FH]
    ghi_ref[...] = g1[:, FH:]
    dis_ref[...] = dis
    hr = lax.dot_general(xr_ref[...], wroot_ref[...], (((1,), (1,)), ((), ())),
                         preferred_element_type=jnp.float32)
    hr_ref[...] = jnp.maximum(hr + broot_ref[...], 0.0)


_k2 = pl.pallas_call(
    _k2_body,
    grid=(NP // RB,),
    in_specs=[
        pl.BlockSpec((RB, F), lambda i: (i, 0)),
        pl.BlockSpec((H, F), lambda i: (0, 0)),
        pl.BlockSpec((NS, RB), lambda i: (0, i)),
        pl.BlockSpec((G, F), lambda i: (0, 0)),
        pl.BlockSpec((H, F), lambda i: (0, 0)),
        pl.BlockSpec((1, H), lambda i: (0, 0)),
    ],
    out_specs=[
        pl.BlockSpec((RB, FH), lambda i: (i, 0)),
        pl.BlockSpec((RB, FH), lambda i: (i, 0)),
        pl.BlockSpec((RB, 1), lambda i: (i, 0)),
        pl.BlockSpec((G, H), lambda i: (0, 0)),
    ],
    out_shape=[jax.ShapeDtypeStruct((NP, FH), jnp.float32),
               jax.ShapeDtypeStruct((NP, FH), jnp.float32),
               jax.ShapeDtypeStruct((NP, 1), jnp.float32),
               jax.ShapeDtypeStruct((G, H), jnp.float32)],
)


def _k4_body(acclo_ref, acchi_ref, dis_ref, b1_ref, w2_ref, glo_ref, ghi_ref):
    dis = dis_ref[...]
    cat = jnp.concatenate([acclo_ref[...], acchi_ref[...]], axis=-1)
    h1 = jnp.maximum(cat * dis + b1_ref[...], 0.0)
    m = lax.dot_general(h1, w2_ref[...], (((1,), (1,)), ((), ())),
                        preferred_element_type=jnp.float32)
    g2 = m * dis
    glo_ref[...] = g2[:, :FH]
    ghi_ref[...] = g2[:, FH:]


_k4 = pl.pallas_call(
    _k4_body,
    grid=(NP // RB,),
    in_specs=[
        pl.BlockSpec((RB, FH), lambda i: (i, 0)),
        pl.BlockSpec((RB, FH), lambda i: (i, 0)),
        pl.BlockSpec((RB, 1), lambda i: (i, 0)),
        pl.BlockSpec((1, H), lambda i: (0, 0)),
        pl.BlockSpec((H, H), lambda i: (0, 0)),
    ],
    out_specs=[pl.BlockSpec((RB, FH), lambda i: (i, 0)),
               pl.BlockSpec((RB, FH), lambda i: (i, 0))],
    out_shape=[jax.ShapeDtypeStruct((NP, FH), jnp.float32),
               jax.ShapeDtypeStruct((NP, FH), jnp.float32)],
)


def _k6_body(pool_ref, hr_ref, wout_ref, bout_ref, out_ref):
    p = pool_ref[...]
    hcat = jnp.concatenate([p[0], p[1], hr_ref[...]], axis=-1)
    out = lax.dot_general(hcat, wout_ref[...], (((1,), (1,)), ((), ())),
                          preferred_element_type=jnp.float32)
    out_ref[...] = out + bout_ref[...]


_k6 = pl.pallas_call(
    _k6_body,
    out_shape=jax.ShapeDtypeStruct((G, C), jnp.float32),
)


def kernel(x, edge_index, batch, W1, b1, W2, b2, Wroot, broot, Wout, bout):
    i32 = jnp.int32
    x_pad = jnp.pad(x, ((0, NP - N), (0, 0)))
    batch_pad = jnp.concatenate([batch, jnp.full((NP - N,), G, i32)])
    pad_e = EPAD - E
    src2d = jnp.concatenate(
        [edge_index[0], jnp.full((pad_e,), N, i32)]).reshape(NS * CPT, CH)
    dst2d = jnp.concatenate(
        [edge_index[1], jnp.full((pad_e,), N, i32)]).reshape(NS * CPT, CH)

    degp, xr = _k1(edge_index[1], batch, x)
    g1lo, g1hi, dis, hr = _k2(x_pad, W1, degp.reshape(NS, NP), xr, Wroot,
                              broot.reshape(1, H))
    acc1lo, acc1hi = _k3(g1lo, g1hi, src2d, dst2d)
    g2lo, g2hi = _k4(acc1lo, acc1hi, dis, b1.reshape(1, H), W2)
    (poolf,) = _k5(g2lo, g2hi, src2d, dst2d, batch_pad, dis.reshape(NP),
                   b2.reshape(NC, 1, FH))
    out = _k6(poolf.reshape(NC, G, FH), hr, Wout, bout.reshape(1, C))
    return out


# speedup vs baseline: 1.0404x; 1.0404x over previous
import jax
import jax.numpy as jnp
from jax import lax
from jax.experimental import pallas as pl
from jax.experimental.pallas import tpu as pltpu
from jax.experimental.pallas import tpu_sc as plsc

N, E, F, H, C, G = 10000, 320000, 128, 128, 2, 128
NC, NS, LANES = 2, 16, 16
NP = 10240
RPT = NP // NS
FH = H // NC
CH = 96
CPT = 209
EPAD = NS * CPT * CH
GP = G + 8
MW = (G * FH) // NS
RB = 2048
PCH = 40
NBUF = 4
GL = 3
IL = 4
IBUF = 10

_mesh = plsc.VectorSubcoreMesh(
    core_axis_name="c", subcore_axis_name="s", num_cores=NC, num_subcores=NS)


def _k1_body(dst_ref, batch_ref, x_ref, degp_out, xr_out,
             dstbuf, degbuf, batchbuf, histbuf, rootsbuf, xrbuf):
    c = lax.axis_index("c")
    s = lax.axis_index("s")
    ept = E // NS

    @pl.when(c == 0)
    def _():
        def zero_deg(i, carry):
            degbuf[pl.ds(i * LANES, LANES)] = jnp.zeros((LANES,), jnp.float32)
            return carry
        lax.fori_loop(0, NP // LANES, zero_deg, 0)

        pltpu.sync_copy(dst_ref.at[pl.ds(s * ept, ept)], dstbuf)
        ones = jnp.ones((LANES,), jnp.float32)

        def acc_deg(i, carry):
            idx = dstbuf[pl.ds(i * LANES, LANES)]
            plsc.addupdate_scatter(degbuf, [idx], ones)
            return carry
        lax.fori_loop(0, ept // LANES, acc_deg, 0)
        pltpu.sync_copy(degbuf, degp_out.at[pl.ds(s * NP, NP)])

    @pl.when((c == 1) & (s == 0))
    def _():
        for k in range(G // LANES):
            histbuf[pl.ds(k * LANES, LANES)] = jnp.zeros((LANES,), jnp.int32)
        pltpu.sync_copy(batch_ref, batchbuf)
        ones_i = jnp.ones((LANES,), jnp.int32)

        def acc_hist(i, carry):
            idx = batchbuf[pl.ds(i * LANES, LANES)]
            plsc.addupdate_scatter(histbuf, [idx], ones_i)
            return carry
        lax.fori_loop(0, N // LANES, acc_hist, 0)

        carry = jnp.int32(0)
        for k in range(G // LANES):
            v = histbuf[pl.ds(k * LANES, LANES)]
            inc = plsc.cumsum(v)
            excl = inc - v + carry
            rootsbuf[pl.ds(k * LANES, LANES)] = jnp.minimum(excl, N - 1)
            carry = carry + jnp.sum(v)
        pltpu.sync_copy(x_ref.at[rootsbuf], xrbuf)
        pltpu.sync_copy(xrbuf, xr_out)


_k1 = pl.kernel(
    _k1_body,
    out_type=[jax.ShapeDtypeStruct((NS * NP,), jnp.float32),
              jax.ShapeDtypeStruct((G, F), jnp.float32)],
    mesh=_mesh,
    compiler_params=pltpu.CompilerParams(needs_layout_passes=False, use_tc_tiling_on_sc=False),
    scratch_types=[
        pltpu.VMEM((E // NS,), jnp.int32),
        pltpu.VMEM((NP,), jnp.float32),
        pltpu.VMEM((N,), jnp.int32),
        pltpu.VMEM((G,), jnp.int32),
        pltpu.VMEM((G,), jnp.int32),
        pltpu.VMEM((G, F), jnp.float32),
    ],
)


def _prop_core(c, s, glo_ref, ghi_ref, src_ref, dst_ref,
               srcring, dstring, rowsbufs, sg, ss, si, sd, shared_acc,
               shared_g):
    rows = pl.ds(s * RPT, RPT)

    def run_core(gref):
        pltpu.sync_copy(gref.at[rows], shared_acc.at[rows])
        pltpu.sync_copy(gref.at[rows], shared_g.at[rows])
        plsc.subcore_barrier()

        def i_start(j):
            b = lax.rem(j, IBUF)
            pltpu.async_copy(src_ref.at[s * CPT + j], srcring.at[b], si.at[b])
            pltpu.async_copy(dst_ref.at[s * CPT + j], dstring.at[b], sd.at[b])

        def i_wait(j):
            b = lax.rem(j, IBUF)
            pltpu.make_async_copy(
                src_ref.at[s * CPT + j], srcring.at[b], si.at[b]).wait()
            pltpu.make_async_copy(
                dst_ref.at[s * CPT + j], dstring.at[b], sd.at[b]).wait()

        def g_start(j):
            b = lax.rem(j, NBUF)
            bi = lax.rem(j, IBUF)
            pltpu.async_copy(shared_g.at[srcring.at[bi]], rowsbufs.at[b],
                             sg.at[b])

        def g_wait(j):
            b = lax.rem(j, NBUF)
            bi = lax.rem(j, IBUF)
            pltpu.make_async_copy(
                shared_g.at[srcring.at[bi]], rowsbufs.at[b], sg.at[b]).wait()

        def s_start(j):
            b = lax.rem(j, NBUF)
            bi = lax.rem(j, IBUF)
            pltpu.async_copy(rowsbufs.at[b], shared_acc.at[dstring.at[bi]],
                             ss.at[b], add=True)

        def s_wait(j):
            b = lax.rem(j, NBUF)
            bi = lax.rem(j, IBUF)
            pltpu.make_async_copy(
                rowsbufs.at[b], shared_acc.at[dstring.at[bi]], ss.at[b]).wait()

        for j in range(IL):
            i_start(jnp.int32(j))
        for j in range(GL):
            i_wait(jnp.int32(j))
            g_start(jnp.int32(j))

        def step(i, carry):
            @pl.when(i + IL < CPT)
            def _():
                i_start(i + IL)

            @pl.when(i + GL < CPT)
            def _():
                @pl.when(i + GL >= NBUF)
                def _():
                    s_wait(i + GL - NBUF)
                i_wait(i + GL)
                g_start(i + GL)
            g_wait(i)
            s_start(i)
            return carry
        lax.fori_loop(0, CPT, step, 0)
        for k in range(NBUF):
            s_wait(jnp.int32(CPT - NBUF + k))

    @pl.when(c == 0)
    def _():
        run_core(glo_ref)

    @pl.when(c == 1)
    def _():
        run_core(ghi_ref)
    plsc.subcore_barrier()


def _k3_body(glo_ref, ghi_ref, src_ref, dst_ref, acclo_out, acchi_out,
             srcring, dstring, rowsbufs, sg, ss, si, sd, shared_acc,
             shared_g):
    c = lax.axis_index("c")
    s = lax.axis_index("s")
    _prop_core(c, s, glo_ref, ghi_ref, src_ref, dst_ref,
               srcring, dstring, rowsbufs, sg, ss, si, sd, shared_acc,
               shared_g)
    rows = pl.ds(s * RPT, RPT)

    @pl.when(c == 0)
    def _():
        pltpu.sync_copy(shared_acc.at[rows], acclo_out.at[rows])

    @pl.when(c == 1)
    def _():
        pltpu.sync_copy(shared_acc.at[rows], acchi_out.at[rows])


_prop_scratch = [
    pltpu.VMEM((IBUF, CH), jnp.int32),
    pltpu.VMEM((IBUF, CH), jnp.int32),
    pltpu.VMEM((NBUF, CH, FH), jnp.float32),
    pltpu.SemaphoreType.DMA((NBUF,)),
    pltpu.SemaphoreType.DMA((NBUF,)),
    pltpu.SemaphoreType.DMA((IBUF,)),
    pltpu.SemaphoreType.DMA((IBUF,)),
    pltpu.VMEM_SHARED((NP, FH), jnp.float32),
    pltpu.VMEM_SHARED((NP, FH), jnp.float32),
]

_k3 = pl.kernel(
    _k3_body,
    out_type=[jax.ShapeDtypeStruct((NP, FH), jnp.float32),
              jax.ShapeDtypeStruct((NP, FH), jnp.float32)],
    mesh=_mesh,
    compiler_params=pltpu.CompilerParams(needs_layout_passes=False, use_tc_tiling_on_sc=False),
    scratch_types=list(_prop_scratch),
)


def _k5_body(glo_ref, ghi_ref, src_ref, dst_ref, batch_ref, dis_ref, b2_ref,
             pool_out,
             srcring, dstring, rowsbufs, sg, ss, si, sd, shared_acc,
             shared_g,
             chunkbuf, batchv, disv, b2v, poolbuf, mtmp):
    c = lax.axis_index("c")
    s = lax.axis_index("s")
    _prop_core(c, s, glo_ref, ghi_ref, src_ref, dst_ref,
               srcring, dstring, rowsbufs, sg, ss, si, sd, shared_acc,
               shared_g)

    rows = pl.ds(s * RPT, RPT)
    pltpu.sync_copy(batch_ref.at[rows], batchv)
    pltpu.sync_copy(dis_ref.at[rows], disv)
    pltpu.sync_copy(b2_ref.at[c], b2v)
    ninf = jnp.full((LANES,), -jnp.inf, jnp.float32)

    def zero_pool(r, carry):
        for j in range(FH // LANES):
            poolbuf[r, pl.ds(j * LANES, LANES)] = ninf
        return carry
    lax.fori_loop(0, GP, zero_pool, 0)

    iota = lax.iota(jnp.int32, LANES)
    zi = jnp.zeros((LANES,), jnp.int32)

    def pool_chunk(cc, carry):
        pltpu.sync_copy(shared_acc.at[pl.ds(s * RPT + cc * PCH, PCH)], chunkbuf)

        def row_step(rr, rcarry):
            r = cc * PCH + rr
            rsplat = zi + r
            bvec = plsc.load_gather(batchv, [rsplat])
            dvec = plsc.load_gather(disv, [rsplat])
            for j in range(FH // LANES):
                v = chunkbuf[rr, pl.ds(j * LANES, LANES)]
                h = jnp.maximum(v * dvec + b2v[0, pl.ds(j * LANES, LANES)], 0.0)
                col = (j * LANES) + iota
                old = plsc.load_gather(poolbuf, [bvec, col])
                plsc.store_scatter(poolbuf, [bvec, col], jnp.maximum(old, h))
            return rcarry
        lax.fori_loop(0, PCH, row_step, 0)
        return carry
    lax.fori_loop(0, RPT // PCH, pool_chunk, 0)

    pltpu.sync_copy(poolbuf.at[pl.ds(0, G)], shared_g.at[pl.ds(s * G, G)])
    plsc.subcore_barrier()
    gsl = pl.ds(s * (G // NS), G // NS)
    pltpu.sync_copy(shared_g.at[pl.ds(0 * G + s * (G // NS), G // NS)], mtmp)
    acc_vecs = [mtmp[r, pl.ds(j * LANES, LANES)]
                for r in range(G // NS) for j in range(FH // LANES)]
    for k in range(1, NS):
        pltpu.sync_copy(shared_g.at[pl.ds(k * G + s * (G // NS), G // NS)],
                        mtmp)
        vi = 0
        for r in range(G // NS):
            for j in range(FH // LANES):
                acc_vecs[vi] = jnp.maximum(
                    acc_vecs[vi], mtmp[r, pl.ds(j * LANES, LANES)])
                vi += 1
    vi = 0
    for r in range(G // NS):
        for j in range(FH // LANES):
            mtmp[r, pl.ds(j * LANES, LANES)] = acc_vecs[vi]
            vi += 1
    pltpu.sync_copy(mtmp, pool_out.at[pl.ds(c * G + s * (G // NS), G // NS)])


_k5 = pl.kernel(
    _k5_body,
    out_type=[jax.ShapeDtypeStruct((NC * G, FH), jnp.float32)],
    mesh=_mesh,
    compiler_params=pltpu.CompilerParams(needs_layout_passes=False, use_tc_tiling_on_sc=False),
    scratch_types=list(_prop_scratch) + [
        pltpu.VMEM((PCH, FH), jnp.float32),
        pltpu.VMEM((RPT,), jnp.int32),
        pltpu.VMEM((RPT,), jnp.float32),
        pltpu.VMEM((1, FH), jnp.float32),
        pltpu.VMEM((GP, FH), jnp.float32),
        pltpu.VMEM((G // NS, FH), jnp.float32),
    ],
)


def _k2_body(x_ref, w1_ref, degp_ref, xr_ref, wroot_ref, broot_ref,
             glo_ref, ghi_ref, dis_ref, hr_ref):
    m = lax.dot_general(x_ref[...], w1_ref[...], (((1,), (1,)), ((), ())),
                        preferred_element_type=jnp.float32)
    deg = lax.dot_general(degp_ref[...], jnp.ones((NS, 1), jnp.float32),
                          (((0,), (0,)), ((), ())),
                          preferred_element_type=jnp.float32)
    dis = lax.rsqrt(deg + 1.0)
    g1 = m * dis
    glo_ref[...] = g1[:, :FH]
    ghi_ref[...] = g1[:, FH:]
    dis_ref[...] = dis
    hr = lax.dot_general(xr_ref[...], wroot_ref[...], (((1,), (1,)), ((), ())),
                         preferred_element_type=jnp.float32)
    hr_ref[...] = jnp.maximum(hr + broot_ref[...], 0.0)


_k2 = pl.pallas_call(
    _k2_body,
    grid=(NP // RB,),
    in_specs=[
        pl.BlockSpec((RB, F), lambda i: (i, 0)),
        pl.BlockSpec((H, F), lambda i: (0, 0)),
        pl.BlockSpec((NS, RB), lambda i: (0, i)),
        pl.BlockSpec((G, F), lambda i: (0, 0)),
        pl.BlockSpec((H, F), lambda i: (0, 0)),
        pl.BlockSpec((1, H), lambda i: (0, 0)),
    ],
    out_specs=[
        pl.BlockSpec((RB, FH), lambda i: (i, 0)),
        pl.BlockSpec((RB, FH), lambda i: (i, 0)),
        pl.BlockSpec((RB, 1), lambda i: (i, 0)),
        pl.BlockSpec((G, H), lambda i: (0, 0)),
    ],
    out_shape=[jax.ShapeDtypeStruct((NP, FH), jnp.float32),
               jax.ShapeDtypeStruct((NP, FH), jnp.float32),
               jax.ShapeDtypeStruct((NP, 1), jnp.float32),
               jax.ShapeDtypeStruct((G, H), jnp.float32)],
)


def _k4_body(acclo_ref, acchi_ref, dis_ref, b1_ref, w2_ref, glo_ref, ghi_ref):
    dis = dis_ref[...]
    cat = jnp.concatenate([acclo_ref[...], acchi_ref[...]], axis=-1)
    h1 = jnp.maximum(cat * dis + b1_ref[...], 0.0)
    m = lax.dot_general(h1, w2_ref[...], (((1,), (1,)), ((), ())),
                        preferred_element_type=jnp.float32)
    g2 = m * dis
    glo_ref[...] = g2[:, :FH]
    ghi_ref[...] = g2[:, FH:]


_k4 = pl.pallas_call(
    _k4_body,
    grid=(NP // RB,),
    in_specs=[
        pl.BlockSpec((RB, FH), lambda i: (i, 0)),
        pl.BlockSpec((RB, FH), lambda i: (i, 0)),
        pl.BlockSpec((RB, 1), lambda i: (i, 0)),
        pl.BlockSpec((1, H), lambda i: (0, 0)),
        pl.BlockSpec((H, H), lambda i: (0, 0)),
    ],
    out_specs=[pl.BlockSpec((RB, FH), lambda i: (i, 0)),
               pl.BlockSpec((RB, FH), lambda i: (i, 0))],
    out_shape=[jax.ShapeDtypeStruct((NP, FH), jnp.float32),
               jax.ShapeDtypeStruct((NP, FH), jnp.float32)],
)


def _k6_body(pool_ref, hr_ref, wout_ref, bout_ref, out_ref):
    p = pool_ref[...]
    hcat = jnp.concatenate([p[0], p[1], hr_ref[...]], axis=-1)
    out = lax.dot_general(hcat, wout_ref[...], (((1,), (1,)), ((), ())),
                          preferred_element_type=jnp.float32)
    out_ref[...] = out + bout_ref[...]


_k6 = pl.pallas_call(
    _k6_body,
    out_shape=jax.ShapeDtypeStruct((G, C), jnp.float32),
)


def kernel(x, edge_index, batch, W1, b1, W2, b2, Wroot, broot, Wout, bout):
    i32 = jnp.int32
    x_pad = jnp.pad(x, ((0, NP - N), (0, 0)))
    batch_pad = jnp.concatenate([batch, jnp.full((NP - N,), G, i32)])
    pad_e = EPAD - E
    src2d = jnp.concatenate(
        [edge_index[0], jnp.full((pad_e,), N, i32)]).reshape(NS * CPT, CH)
    dst2d = jnp.concatenate(
        [edge_index[1], jnp.full((pad_e,), N, i32)]).reshape(NS * CPT, CH)

    degp, xr = _k1(edge_index[1], batch, x)
    g1lo, g1hi, dis, hr = _k2(x_pad, W1, degp.reshape(NS, NP), xr, Wroot,
                              broot.reshape(1, H))
    acc1lo, acc1hi = _k3(g1lo, g1hi, src2d, dst2d)
    g2lo, g2hi = _k4(acc1lo, acc1hi, dis, b1.reshape(1, H), W2)
    (poolf,) = _k5(g2lo, g2hi, src2d, dst2d, batch_pad, dis.reshape(NP),
                   b2.reshape(NC, 1, FH))
    out = _k6(poolf.reshape(NC, G, FH), hr, Wout, bout.reshape(1, C))
    return out
